# baseline (speedup 1.0000x reference)
import functools

import jax
import jax.numpy as jnp
from jax import lax
from jax.experimental import pallas as pl
from jax.experimental.pallas import tpu as pltpu
from jax.experimental.pallas import tpu_sc as plsc

_N = 10000
_D = 128
_E = 320000
_EP = 100000
_NC, _NS = 2, 16
_NW = _NC * _NS
_K = 128
_GK = 16
_TCHA = 2560
_A0, _A1 = 32, 128
_D0, _D1 = 80, 80
_NACC = 10112
_RPT = _NACC // _NS
_CS = (128, 128, 128, 128, 120)
_PGK = 8
_P0, _P1 = 40, 64
_PMAX = max(_P0, _P1)
_TCHP = _NS * (_P0 + _P1)
_TCHP_PAD = 1696
_PTOT = _TCHP * _K
_EPPAD = (_TCHP // 2) * _K


def _mesh():
    return plsc.VectorSubcoreMesh(core_axis_name="c", subcore_axis_name="s")


def _core_split(c, s, n0, n1):
    start = jnp.where(c == 0, s * n0, _NS * n0 + s * n1)
    groups = jnp.where(c == 0, n0 // _GK, n1 // _GK)
    return start, groups


def _agg_body(table_h, src_h, dst_h, z_h, part_h,
              src_v, dst_v, rows0, rows1, acc_s,
              gsem0, gsem1, ssem0, ssem1):
    rows = (rows0, rows1)
    gsem = (gsem0, gsem1)
    ssem = (ssem0, ssem1)
    c = lax.axis_index("c")
    s = lax.axis_index("s")
    r0 = s * _RPT
    start, groups = _core_split(c, s, _A0, _A1)
    pltpu.sync_copy(z_h, rows0)
    off = 0
    for n in _CS:
        pltpu.sync_copy(rows0.at[pl.ds(0, n)], acc_s.at[pl.ds(r0 + off, n)])
        off += n
    plsc.subcore_barrier()

    def group(g, carry):
        j0 = start + g * _GK
        pltpu.sync_copy(src_h.at[pl.ds(j0, _GK)], src_v)
        pltpu.sync_copy(dst_h.at[pl.ds(j0, _GK)], dst_v)
        gcp = [None] * _GK
        scp = [None] * _GK
        for t in range(_GK):
            b = t % 2
            if t >= 2:
                scp[t - 2].wait()
            gcp[t] = pltpu.async_copy(
                table_h.at[src_v.at[t]], rows[b], gsem[b])
            if t >= 1:
                p = (t - 1) % 2
                gcp[t - 1].wait()
                scp[t - 1] = pltpu.async_copy(
                    rows[p], acc_s.at[dst_v.at[t - 1]], ssem[p], add=True)
        last = _GK - 1
        gcp[last].wait()
        scp[last] = pltpu.async_copy(
            rows[last % 2], acc_s.at[dst_v.at[last]], ssem[last % 2], add=True)
        scp[last - 1].wait()
        scp[last].wait()
        return carry

    lax.fori_loop(0, groups, group, 0)
    plsc.subcore_barrier()
    off = 0
    for n in _CS:
        pltpu.sync_copy(acc_s.at[pl.ds(r0 + off, n)], rows0.at[pl.ds(0, n)])
        pltpu.sync_copy(rows0.at[pl.ds(0, n)], part_h.at[c, pl.ds(r0 + off, n)])
        off += n


def _sage_agg(table, srcp, dstp, z):
    out_type = jax.ShapeDtypeStruct((_NC, _NACC, _D), jnp.float32)
    scratch = [
        pltpu.VMEM((_GK, _K), jnp.int32),
        pltpu.VMEM((_GK, _K), jnp.int32),
        pltpu.VMEM((_K, _D), jnp.float32),
        pltpu.VMEM((_K, _D), jnp.float32),
        pltpu.VMEM_SHARED((_NACC, _D), jnp.float32),
    ] + [pltpu.SemaphoreType.DMA] * 4
    f = pl.kernel(_agg_body, out_type=out_type, mesh=_mesh(),
                  scratch_types=scratch)
    return f(table, srcp, dstp, z)


def _deg_body(dst_h, z_h, ones_h, part_h,
              dst_v, rows0, acc_s, ssem0, ssem1):
    ssem = (ssem0, ssem1)
    c = lax.axis_index("c")
    s = lax.axis_index("s")
    r0 = s * _RPT
    start, groups = _core_split(c, s, _D0, _D1)
    pltpu.sync_copy(z_h, rows0)
    off = 0
    for n in _CS:
        pltpu.sync_copy(rows0.at[pl.ds(0, n)], acc_s.at[pl.ds(r0 + off, n)])
        off += n
    pltpu.sync_copy(ones_h, rows0)
    plsc.subcore_barrier()

    def group(g, carry):
        j0 = start + g * _GK
        pltpu.sync_copy(dst_h.at[pl.ds(j0, _GK)], dst_v)
        scp = [None] * _GK
        for t in range(_GK):
            if t >= 2:
                scp[t - 2].wait()
            scp[t] = pltpu.async_copy(
                rows0, acc_s.at[dst_v.at[t]], ssem[t % 2], add=True)
        scp[_GK - 2].wait()
        scp[_GK - 1].wait()
        return carry

    lax.fori_loop(0, groups, group, 0)
    plsc.subcore_barrier()
    off = 0
    for n in _CS:
        pltpu.sync_copy(acc_s.at[pl.ds(r0 + off, n)], rows0.at[pl.ds(0, n)])
        pltpu.sync_copy(rows0.at[pl.ds(0, n)], part_h.at[c, pl.ds(r0 + off, n)])
        off += n


def _deg_agg(dstp, z, ones_k):
    out_type = jax.ShapeDtypeStruct((_NC, _NACC, _D), jnp.float32)
    scratch = [
        pltpu.VMEM((_GK, _K), jnp.int32),
        pltpu.VMEM((_K, _D), jnp.float32),
        pltpu.VMEM_SHARED((_NACC, _D), jnp.float32),
    ] + [pltpu.SemaphoreType.DMA] * 2
    f = pl.kernel(_deg_body, out_type=out_type, mesh=_mesh(),
                  scratch_types=scratch)
    return f(dstp, z, ones_k)


def _pair_gather_body(h_h, src_h, dst_h, a_h, b_h, src_v, dst_v,
                      av0, av1, bv0, bv1,
                      gsa0, gsa1, gsb0, gsb1, wsa0, wsa1, wsb0, wsb1):
    av = (av0, av1)
    bv = (bv0, bv1)
    gsa = (gsa0, gsa1)
    gsb = (gsb0, gsb1)
    wsa = (wsa0, wsa1)
    wsb = (wsb0, wsb1)
    c = lax.axis_index("c")
    s = lax.axis_index("s")
    start = jnp.where(c == 0, s * _P0, _NS * _P0 + s * _P1)
    groups = jnp.where(c == 0, _P0 // _PGK, _P1 // _PGK)
    pltpu.sync_copy(src_h.at[pl.ds(start, _PMAX)], src_v)
    pltpu.sync_copy(dst_h.at[pl.ds(start, _PMAX)], dst_v)

    def group(g, carry):
        j0 = g * _PGK
        ga = [None] * _PGK
        gb = [None] * _PGK
        wa = [None] * _PGK
        wb = [None] * _PGK
        for t in range(_PGK):
            b = t % 2
            if t >= 2:
                wa[t - 2].wait()
                wb[t - 2].wait()
            ga[t] = pltpu.async_copy(h_h.at[src_v.at[j0 + t]], av[b], gsa[b])
            gb[t] = pltpu.async_copy(h_h.at[dst_v.at[j0 + t]], bv[b], gsb[b])
            if t >= 1:
                p = (t - 1) % 2
                row = (start + j0 + t - 1) * _K
                ga[t - 1].wait()
                wa[t - 1] = pltpu.async_copy(
                    av[p], a_h.at[pl.ds(row, _K)], wsa[p])
                gb[t - 1].wait()
                wb[t - 1] = pltpu.async_copy(
                    bv[p], b_h.at[pl.ds(row, _K)], wsb[p])
        last = _PGK - 1
        p = last % 2
        row = (start + j0 + last) * _K
        ga[last].wait()
        wa[last] = pltpu.async_copy(av[p], a_h.at[pl.ds(row, _K)], wsa[p])
        gb[last].wait()
        wb[last] = pltpu.async_copy(bv[p], b_h.at[pl.ds(row, _K)], wsb[p])
        wa[last - 1].wait()
        wb[last - 1].wait()
        wa[last].wait()
        wb[last].wait()
        return carry

    lax.fori_loop(0, groups, group, 0)


def _pair_gather(h, srcp, dstp):
    out_type = [
        jax.ShapeDtypeStruct((_PTOT, _D), jnp.float32),
        jax.ShapeDtypeStruct((_PTOT, _D), jnp.float32),
    ]
    scratch = [
        pltpu.VMEM((_PMAX, _K), jnp.int32),
        pltpu.VMEM((_PMAX, _K), jnp.int32),
        pltpu.VMEM((_K, _D), jnp.float32),
        pltpu.VMEM((_K, _D), jnp.float32),
        pltpu.VMEM((_K, _D), jnp.float32),
        pltpu.VMEM((_K, _D), jnp.float32),
    ] + [pltpu.SemaphoreType.DMA] * 8
    f = pl.kernel(_pair_gather_body, out_type=out_type, mesh=_mesh(),
                  scratch_types=scratch)
    return f(h, srcp, dstp)


def _layer_tc(h, parts, degp, Ws, Wn, b, relu):
    R = 1000

    def body(h_ref, p_ref, d_ref, ws_ref, wn_ref, b_ref, o_ref):
        deg = jnp.maximum(d_ref[0, :, 0] + d_ref[1, :, 0], 1.0)
        agg = (p_ref[0] + p_ref[1]) / deg[:, None]
        o = (jnp.dot(h_ref[...], ws_ref[...], preferred_element_type=jnp.float32)
             + jnp.dot(agg, wn_ref[...], preferred_element_type=jnp.float32)
             + b_ref[...])
        if relu:
            o = jnp.maximum(o, 0.0)
        o_ref[...] = o

    return pl.pallas_call(
        body,
        grid=(_N // R,),
        in_specs=[
            pl.BlockSpec((R, _D), lambda i: (i, 0)),
            pl.BlockSpec((_NC, R, _D), lambda i: (0, i, 0)),
            pl.BlockSpec((_NC, R, _D), lambda i: (0, i, 0)),
            pl.BlockSpec((_D, _D), lambda i: (0, 0)),
            pl.BlockSpec((_D, _D), lambda i: (0, 0)),
            pl.BlockSpec((1, _D), lambda i: (0, 0)),
        ],
        out_specs=pl.BlockSpec((R, _D), lambda i: (i, 0)),
        out_shape=jax.ShapeDtypeStruct((_N, _D), jnp.float32),
    )(h, parts, degp, Ws, Wn, b.reshape(1, _D))


def _mlp_tc(a, b, P0, pb0, P1, pb1, P2, pb2):
    R = 1024

    def body(a_ref, b_ref, p0, q0, p1, q1, p2, q2, o_ref):
        z = a_ref[...] * b_ref[...]
        z = jnp.maximum(
            jnp.dot(z, p0[...], preferred_element_type=jnp.float32) + q0[...], 0.0)
        z = jnp.maximum(
            jnp.dot(z, p1[...], preferred_element_type=jnp.float32) + q1[...], 0.0)
        o_ref[...] = jnp.dot(z, p2[...], preferred_element_type=jnp.float32) + q2[...]

    return pl.pallas_call(
        body,
        grid=(_PTOT // R,),
        in_specs=[
            pl.BlockSpec((R, _D), lambda i: (i, 0)),
            pl.BlockSpec((R, _D), lambda i: (i, 0)),
            pl.BlockSpec((_D, _D), lambda i: (0, 0)),
            pl.BlockSpec((1, _D), lambda i: (0, 0)),
            pl.BlockSpec((_D, _D), lambda i: (0, 0)),
            pl.BlockSpec((1, _D), lambda i: (0, 0)),
            pl.BlockSpec((_D, 1), lambda i: (0, 0)),
            pl.BlockSpec((1, 1), lambda i: (0, 0)),
        ],
        out_specs=pl.BlockSpec((R, 1), lambda i: (i, 0)),
        out_shape=jax.ShapeDtypeStruct((_PTOT, 1), jnp.float32),
    )(a, b, P0, pb0.reshape(1, _D), P1, pb1.reshape(1, _D), P2,
      pb2.reshape(1, 1))


def kernel(x, edge_index, pos_src, pos_dst, neg_src, neg_dst,
           W_self0, W_neigh0, b0, W_self1, W_neigh1, b1, W_self2, W_neigh2, b2,
           P0, pb0, P1, pb1, P2, pb2):
    src = edge_index[0]
    dst = edge_index[1]
    pad_e = _TCHA * _K - _E
    srcp = jnp.concatenate(
        [src, jnp.zeros((pad_e,), jnp.int32)]).reshape(_TCHA, _K)
    dstp = jnp.concatenate(
        [dst, jnp.full((pad_e,), _N, jnp.int32)]).reshape(_TCHA, _K)
    z = jnp.zeros((_K, _D), jnp.float32)
    ones_k = jnp.ones((_K, _D), jnp.float32)

    degp = _deg_agg(dstp, z, ones_k)

    parts = _sage_agg(x, srcp, dstp, z)
    h1 = _layer_tc(x, parts, degp, W_self0, W_neigh0, b0, relu=True)
    parts = _sage_agg(h1, srcp, dstp, z)
    h2 = _layer_tc(h1, parts, degp, W_self1, W_neigh1, b1, relu=True)
    parts = _sage_agg(h2, srcp, dstp, z)
    h3 = _layer_tc(h2, parts, degp, W_self2, W_neigh2, b2, relu=False)

    pad_p = _EPPAD - _EP
    zp = jnp.zeros((pad_p,), jnp.int32)
    tail = jnp.zeros(((_TCHP_PAD - _TCHP) * _K,), jnp.int32)
    ps = jnp.concatenate(
        [pos_src, zp, neg_src, zp, tail]).reshape(_TCHP_PAD, _K)
    pd = jnp.concatenate(
        [pos_dst, zp, neg_dst, zp, tail]).reshape(_TCHP_PAD, _K)
    a, bm = _pair_gather(h3, ps, pd)
    scores = _mlp_tc(a, bm, P0, pb0, P1, pb1, P2, pb2)
    return scores[:_EP], scores[_EPPAD:_EPPAD + _EP]


# speedup vs baseline: 1.5757x; 1.5757x over previous
import functools

import jax
import jax.numpy as jnp
from jax import lax
from jax.experimental import pallas as pl
from jax.experimental.pallas import tpu as pltpu
from jax.experimental.pallas import tpu_sc as plsc

_N = 10000
_D = 128
_E = 320000
_EP = 100000
_NC, _NS = 2, 16
_NW = _NC * _NS
_K = 128
_GK = 16
_G = 5
_CH = _G * _GK
_NACC = 10112
_RPT = _NACC // _NS
_CS = (128, 128, 128, 128, 120)
_PGK = 7
_CHP = 49
_EPPAD = (_NW // 2) * _CHP * _K
_PTOT = 2 * _EPPAD


def _mesh():
    return plsc.VectorSubcoreMesh(core_axis_name="c", subcore_axis_name="s")


def _agg_body(table_h, src_h, dst_h, z_h, part_h,
              src_v, dst_v, rows0, rows1, acc_s,
              gsem0, gsem1, ssem0, ssem1):
    rows = (rows0, rows1)
    gsem = (gsem0, gsem1)
    ssem = (ssem0, ssem1)
    c = lax.axis_index("c")
    s = lax.axis_index("s")
    wid = s * _NC + c
    r0 = s * _RPT
    pltpu.sync_copy(z_h, rows0)
    off = 0
    for n in _CS:
        pltpu.sync_copy(rows0.at[pl.ds(0, n)], acc_s.at[pl.ds(r0 + off, n)])
        off += n
    plsc.subcore_barrier()

    def group(g, carry):
        pltpu.sync_copy(src_h.at[wid, pl.ds(g * _GK, _GK)], src_v)
        pltpu.sync_copy(dst_h.at[wid, pl.ds(g * _GK, _GK)], dst_v)
        gcp = [None] * _GK
        scp = [None] * _GK
        for t in range(_GK):
            b = t % 2
            if t >= 2:
                scp[t - 2].wait()
            gcp[t] = pltpu.async_copy(
                table_h.at[src_v.at[t]], rows[b], gsem[b])
            if t >= 1:
                p = (t - 1) % 2
                gcp[t - 1].wait()
                scp[t - 1] = pltpu.async_copy(
                    rows[p], acc_s.at[dst_v.at[t - 1]], ssem[p], add=True)
        last = _GK - 1
        gcp[last].wait()
        scp[last] = pltpu.async_copy(
            rows[last % 2], acc_s.at[dst_v.at[last]], ssem[last % 2], add=True)
        scp[last - 1].wait()
        scp[last].wait()
        return carry

    lax.fori_loop(0, _G, group, 0)
    plsc.subcore_barrier()
    off = 0
    for n in _CS:
        pltpu.sync_copy(acc_s.at[pl.ds(r0 + off, n)], rows0.at[pl.ds(0, n)])
        pltpu.sync_copy(rows0.at[pl.ds(0, n)], part_h.at[c, pl.ds(r0 + off, n)])
        off += n


def _sage_agg(table, srcp, dstp, z):
    out_type = jax.ShapeDtypeStruct((_NC, _NACC, _D), jnp.float32)
    scratch = [
        pltpu.VMEM((_GK, _K), jnp.int32),
        pltpu.VMEM((_GK, _K), jnp.int32),
        pltpu.VMEM((_K, _D), jnp.float32),
        pltpu.VMEM((_K, _D), jnp.float32),
        pltpu.VMEM_SHARED((_NACC, _D), jnp.float32),
    ] + [pltpu.SemaphoreType.DMA] * 4
    f = pl.kernel(_agg_body, out_type=out_type, mesh=_mesh(),
                  scratch_types=scratch)
    return f(table, srcp, dstp, z)


def _deg_body(dst_h, z_h, ones_h, part_h,
              dst_v, rows0, acc_s, ssem0, ssem1):
    ssem = (ssem0, ssem1)
    c = lax.axis_index("c")
    s = lax.axis_index("s")
    wid = s * _NC + c
    r0 = s * _RPT
    pltpu.sync_copy(z_h, rows0)
    off = 0
    for n in _CS:
        pltpu.sync_copy(rows0.at[pl.ds(0, n)], acc_s.at[pl.ds(r0 + off, n)])
        off += n
    pltpu.sync_copy(ones_h, rows0)
    plsc.subcore_barrier()

    def group(g, carry):
        pltpu.sync_copy(dst_h.at[wid, pl.ds(g * _GK, _GK)], dst_v)
        scp = [None] * _GK
        for t in range(_GK):
            if t >= 2:
                scp[t - 2].wait()
            scp[t] = pltpu.async_copy(
                rows0, acc_s.at[dst_v.at[t]], ssem[t % 2], add=True)
        scp[_GK - 2].wait()
        scp[_GK - 1].wait()
        return carry

    lax.fori_loop(0, _G, group, 0)
    plsc.subcore_barrier()
    off = 0
    for n in _CS:
        pltpu.sync_copy(acc_s.at[pl.ds(r0 + off, n)], rows0.at[pl.ds(0, n)])
        pltpu.sync_copy(rows0.at[pl.ds(0, n)], part_h.at[c, pl.ds(r0 + off, n)])
        off += n


def _deg_agg(dstp, z, ones_k):
    out_type = jax.ShapeDtypeStruct((_NC, _NACC, _D), jnp.float32)
    scratch = [
        pltpu.VMEM((_GK, _K), jnp.int32),
        pltpu.VMEM((_K, _D), jnp.float32),
        pltpu.VMEM_SHARED((_NACC, _D), jnp.float32),
    ] + [pltpu.SemaphoreType.DMA] * 2
    f = pl.kernel(_deg_body, out_type=out_type, mesh=_mesh(),
                  scratch_types=scratch)
    return f(dstp, z, ones_k)


def _pair_gather_body(h_h, src_h, dst_h, a_h, b_h, src_v, dst_v,
                      av0, av1, bv0, bv1,
                      gsa0, gsa1, gsb0, gsb1, wsa0, wsa1, wsb0, wsb1):
    av = (av0, av1)
    bv = (bv0, bv1)
    gsa = (gsa0, gsa1)
    gsb = (gsb0, gsb1)
    wsa = (wsa0, wsa1)
    wsb = (wsb0, wsb1)
    c = lax.axis_index("c")
    s = lax.axis_index("s")
    wid = s * _NC + c
    base = wid * _CHP
    pltpu.sync_copy(src_h.at[wid], src_v)
    pltpu.sync_copy(dst_h.at[wid], dst_v)

    def group(g, carry):
        j0 = g * _PGK
        ga = [None] * _PGK
        gb = [None] * _PGK
        wa = [None] * _PGK
        wb = [None] * _PGK
        for t in range(_PGK):
            b = t % 2
            if t >= 2:
                wa[t - 2].wait()
                wb[t - 2].wait()
            ga[t] = pltpu.async_copy(h_h.at[src_v.at[j0 + t]], av[b], gsa[b])
            gb[t] = pltpu.async_copy(h_h.at[dst_v.at[j0 + t]], bv[b], gsb[b])
            if t >= 1:
                p = (t - 1) % 2
                row = (base + j0 + t - 1) * _K
                ga[t - 1].wait()
                wa[t - 1] = pltpu.async_copy(
                    av[p], a_h.at[pl.ds(row, _K)], wsa[p])
                gb[t - 1].wait()
                wb[t - 1] = pltpu.async_copy(
                    bv[p], b_h.at[pl.ds(row, _K)], wsb[p])
        last = _PGK - 1
        p = last % 2
        row = (base + j0 + last) * _K
        ga[last].wait()
        wa[last] = pltpu.async_copy(av[p], a_h.at[pl.ds(row, _K)], wsa[p])
        gb[last].wait()
        wb[last] = pltpu.async_copy(bv[p], b_h.at[pl.ds(row, _K)], wsb[p])
        wa[last - 1].wait()
        wb[last - 1].wait()
        wa[last].wait()
        wb[last].wait()
        return carry

    lax.fori_loop(0, _CHP // _PGK, group, 0)


def _pair_gather(h, srcp, dstp):
    out_type = [
        jax.ShapeDtypeStruct((_PTOT, _D), jnp.float32),
        jax.ShapeDtypeStruct((_PTOT, _D), jnp.float32),
    ]
    scratch = [
        pltpu.VMEM((_CHP, _K), jnp.int32),
        pltpu.VMEM((_CHP, _K), jnp.int32),
        pltpu.VMEM((_K, _D), jnp.float32),
        pltpu.VMEM((_K, _D), jnp.float32),
        pltpu.VMEM((_K, _D), jnp.float32),
        pltpu.VMEM((_K, _D), jnp.float32),
    ] + [pltpu.SemaphoreType.DMA] * 8
    f = pl.kernel(_pair_gather_body, out_type=out_type, mesh=_mesh(),
                  scratch_types=scratch)
    return f(h, srcp, dstp)


def _layer_tc(h, parts, degp, Ws, Wn, b, relu):
    R = 1000

    def body(h_ref, p_ref, d_ref, ws_ref, wn_ref, b_ref, o_ref):
        deg = jnp.maximum(d_ref[0, :, 0] + d_ref[1, :, 0], 1.0)
        agg = (p_ref[0] + p_ref[1]) / deg[:, None]
        o = (jnp.dot(h_ref[...], ws_ref[...], preferred_element_type=jnp.float32)
             + jnp.dot(agg, wn_ref[...], preferred_element_type=jnp.float32)
             + b_ref[...])
        if relu:
            o = jnp.maximum(o, 0.0)
        o_ref[...] = o

    return pl.pallas_call(
        body,
        grid=(_N // R,),
        in_specs=[
            pl.BlockSpec((R, _D), lambda i: (i, 0)),
            pl.BlockSpec((_NC, R, _D), lambda i: (0, i, 0)),
            pl.BlockSpec((_NC, R, _D), lambda i: (0, i, 0)),
            pl.BlockSpec((_D, _D), lambda i: (0, 0)),
            pl.BlockSpec((_D, _D), lambda i: (0, 0)),
            pl.BlockSpec((1, _D), lambda i: (0, 0)),
        ],
        out_specs=pl.BlockSpec((R, _D), lambda i: (i, 0)),
        out_shape=jax.ShapeDtypeStruct((_N, _D), jnp.float32),
    )(h, parts, degp, Ws, Wn, b.reshape(1, _D))


def _mlp_tc(a, b, P0, pb0, P1, pb1, P2, pb2):
    R = 1024

    def body(a_ref, b_ref, p0, q0, p1, q1, p2, q2, o_ref):
        z = a_ref[...] * b_ref[...]
        z = jnp.maximum(
            jnp.dot(z, p0[...], preferred_element_type=jnp.float32) + q0[...], 0.0)
        z = jnp.maximum(
            jnp.dot(z, p1[...], preferred_element_type=jnp.float32) + q1[...], 0.0)
        o_ref[...] = jnp.dot(z, p2[...], preferred_element_type=jnp.float32) + q2[...]

    return pl.pallas_call(
        body,
        grid=(_PTOT // R,),
        in_specs=[
            pl.BlockSpec((R, _D), lambda i: (i, 0)),
            pl.BlockSpec((R, _D), lambda i: (i, 0)),
            pl.BlockSpec((_D, _D), lambda i: (0, 0)),
            pl.BlockSpec((1, _D), lambda i: (0, 0)),
            pl.BlockSpec((_D, _D), lambda i: (0, 0)),
            pl.BlockSpec((1, _D), lambda i: (0, 0)),
            pl.BlockSpec((_D, 1), lambda i: (0, 0)),
            pl.BlockSpec((1, 1), lambda i: (0, 0)),
        ],
        out_specs=pl.BlockSpec((R, 1), lambda i: (i, 0)),
        out_shape=jax.ShapeDtypeStruct((_PTOT, 1), jnp.float32),
    )(a, b, P0, pb0.reshape(1, _D), P1, pb1.reshape(1, _D), P2,
      pb2.reshape(1, 1))


def kernel(x, edge_index, pos_src, pos_dst, neg_src, neg_dst,
           W_self0, W_neigh0, b0, W_self1, W_neigh1, b1, W_self2, W_neigh2, b2,
           P0, pb0, P1, pb1, P2, pb2):
    src = edge_index[0]
    dst = edge_index[1]
    pad_e = _NW * _CH * _K - _E
    srcp = jnp.concatenate(
        [src, jnp.zeros((pad_e,), jnp.int32)]).reshape(_NW, _CH, _K)
    dstp = jnp.concatenate(
        [dst, jnp.full((pad_e,), _N, jnp.int32)]).reshape(_NW, _CH, _K)
    z = jnp.zeros((_K, _D), jnp.float32)
    ones_k = jnp.ones((_K, _D), jnp.float32)

    degp = _deg_agg(dstp, z, ones_k)

    parts = _sage_agg(x, srcp, dstp, z)
    h1 = _layer_tc(x, parts, degp, W_self0, W_neigh0, b0, relu=True)
    parts = _sage_agg(h1, srcp, dstp, z)
    h2 = _layer_tc(h1, parts, degp, W_self1, W_neigh1, b1, relu=True)
    parts = _sage_agg(h2, srcp, dstp, z)
    h3 = _layer_tc(h2, parts, degp, W_self2, W_neigh2, b2, relu=False)

    pad_p = _EPPAD - _EP
    zp = jnp.zeros((pad_p,), jnp.int32)
    ps = jnp.concatenate(
        [pos_src, zp, neg_src, zp]).reshape(_NW, _CHP, _K)
    pd = jnp.concatenate(
        [pos_dst, zp, neg_dst, zp]).reshape(_NW, _CHP, _K)
    a, bm = _pair_gather(h3, ps, pd)
    scores = _mlp_tc(a, bm, P0, pb0, P1, pb1, P2, pb2)
    return scores[:_EP], scores[_EPPAD:_EPPAD + _EP]


# speedup vs baseline: 1.5887x; 1.0083x over previous
import functools

import jax
import jax.numpy as jnp
from jax import lax
from jax.experimental import pallas as pl
from jax.experimental.pallas import tpu as pltpu
from jax.experimental.pallas import tpu_sc as plsc

_N = 10000
_D = 128
_E = 320000
_EP = 100000
_NC, _NS = 2, 16
_NW = _NC * _NS
_K = 128
_KA = 64
_GK = 16
_GA = 10
_CHA = _GA * _GK
_G = 5
_CH = _G * _GK
_NACC = 10112
_RPT = _NACC // _NS
_CS = (128, 128, 128, 128, 120)
_PGK = 7
_CHP = 49
_EPPAD = (_NW // 2) * _CHP * _K
_PTOT = 2 * _EPPAD


def _mesh():
    return plsc.VectorSubcoreMesh(core_axis_name="c", subcore_axis_name="s")


def _agg_body(table_h, src_h, dst_h, z_h, part_h,
              src_v, dst_v, rows0, rows1, acc_s,
              gsem0, gsem1, gsem2, gsem3, ssem0, ssem1, ssem2, ssem3):
    gsem = (gsem0, gsem1, gsem2, gsem3)
    ssem = (ssem0, ssem1, ssem2, ssem3)

    def buf(i):
        q = i % 4
        return (rows0, rows1)[q // 2].at[pl.ds((q % 2) * _KA, _KA)]

    c = lax.axis_index("c")
    s = lax.axis_index("s")
    wid = s * _NC + c
    r0 = s * _RPT
    pltpu.sync_copy(z_h, rows0)
    off = 0
    for n in _CS:
        pltpu.sync_copy(rows0.at[pl.ds(0, n)], acc_s.at[pl.ds(r0 + off, n)])
        off += n
    plsc.subcore_barrier()

    def group(g, carry):
        pltpu.sync_copy(src_h.at[wid, pl.ds(g * _GK, _GK)], src_v)
        pltpu.sync_copy(dst_h.at[wid, pl.ds(g * _GK, _GK)], dst_v)
        gcp = [None] * _GK
        scp = [None] * _GK
        for t in range(_GK):
            q = t % 4
            if t >= 4:
                scp[t - 4].wait()
            gcp[t] = pltpu.async_copy(
                table_h.at[src_v.at[t]], buf(t), gsem[q])
            if t >= 2:
                gcp[t - 2].wait()
                scp[t - 2] = pltpu.async_copy(
                    buf(t - 2), acc_s.at[dst_v.at[t - 2]],
                    ssem[(t - 2) % 4], add=True)
        for t in (_GK - 2, _GK - 1):
            gcp[t].wait()
            scp[t] = pltpu.async_copy(
                buf(t), acc_s.at[dst_v.at[t]], ssem[t % 4], add=True)
        for t in range(_GK - 4, _GK):
            scp[t].wait()
        return carry

    lax.fori_loop(0, _GA, group, 0)
    plsc.subcore_barrier()
    off = 0
    for n in _CS:
        pltpu.sync_copy(acc_s.at[pl.ds(r0 + off, n)], rows0.at[pl.ds(0, n)])
        pltpu.sync_copy(rows0.at[pl.ds(0, n)], part_h.at[c, pl.ds(r0 + off, n)])
        off += n


def _sage_agg(table, srcp, dstp, z):
    out_type = jax.ShapeDtypeStruct((_NC, _NACC, _D), jnp.float32)
    scratch = [
        pltpu.VMEM((_GK, _KA), jnp.int32),
        pltpu.VMEM((_GK, _KA), jnp.int32),
        pltpu.VMEM((_K, _D), jnp.float32),
        pltpu.VMEM((_K, _D), jnp.float32),
        pltpu.VMEM_SHARED((_NACC, _D), jnp.float32),
    ] + [pltpu.SemaphoreType.DMA] * 8
    f = pl.kernel(_agg_body, out_type=out_type, mesh=_mesh(),
                  scratch_types=scratch)
    return f(table, srcp, dstp, z)


def _deg_body(dst_h, z_h, ones_h, part_h,
              dst_v, rows0, acc_s, ssem0, ssem1):
    ssem = (ssem0, ssem1)
    c = lax.axis_index("c")
    s = lax.axis_index("s")
    wid = s * _NC + c
    r0 = s * _RPT
    pltpu.sync_copy(z_h, rows0)
    off = 0
    for n in _CS:
        pltpu.sync_copy(rows0.at[pl.ds(0, n)], acc_s.at[pl.ds(r0 + off, n)])
        off += n
    pltpu.sync_copy(ones_h, rows0)
    plsc.subcore_barrier()

    def group(g, carry):
        pltpu.sync_copy(dst_h.at[wid, pl.ds(g * _GK, _GK)], dst_v)
        scp = [None] * _GK
        for t in range(_GK):
            if t >= 2:
                scp[t - 2].wait()
            scp[t] = pltpu.async_copy(
                rows0, acc_s.at[dst_v.at[t]], ssem[t % 2], add=True)
        scp[_GK - 2].wait()
        scp[_GK - 1].wait()
        return carry

    lax.fori_loop(0, _G, group, 0)
    plsc.subcore_barrier()
    off = 0
    for n in _CS:
        pltpu.sync_copy(acc_s.at[pl.ds(r0 + off, n)], rows0.at[pl.ds(0, n)])
        pltpu.sync_copy(rows0.at[pl.ds(0, n)], part_h.at[c, pl.ds(r0 + off, n)])
        off += n


def _deg_agg(dstp, z, ones_k):
    out_type = jax.ShapeDtypeStruct((_NC, _NACC, _D), jnp.float32)
    scratch = [
        pltpu.VMEM((_GK, _K), jnp.int32),
        pltpu.VMEM((_K, _D), jnp.float32),
        pltpu.VMEM_SHARED((_NACC, _D), jnp.float32),
    ] + [pltpu.SemaphoreType.DMA] * 2
    f = pl.kernel(_deg_body, out_type=out_type, mesh=_mesh(),
                  scratch_types=scratch)
    return f(dstp, z, ones_k)


def _pair_gather_body(h_h, src_h, dst_h, a_h, b_h, src_v, dst_v,
                      av0, av1, bv0, bv1,
                      gsa0, gsa1, gsb0, gsb1, wsa0, wsa1, wsb0, wsb1):
    av = (av0, av1)
    bv = (bv0, bv1)
    gsa = (gsa0, gsa1)
    gsb = (gsb0, gsb1)
    wsa = (wsa0, wsa1)
    wsb = (wsb0, wsb1)
    c = lax.axis_index("c")
    s = lax.axis_index("s")
    wid = s * _NC + c
    base = wid * _CHP
    pltpu.sync_copy(src_h.at[wid], src_v)
    pltpu.sync_copy(dst_h.at[wid], dst_v)

    def group(g, carry):
        j0 = g * _PGK
        ga = [None] * _PGK
        gb = [None] * _PGK
        wa = [None] * _PGK
        wb = [None] * _PGK
        for t in range(_PGK):
            b = t % 2
            if t >= 2:
                wa[t - 2].wait()
                wb[t - 2].wait()
            ga[t] = pltpu.async_copy(h_h.at[src_v.at[j0 + t]], av[b], gsa[b])
            gb[t] = pltpu.async_copy(h_h.at[dst_v.at[j0 + t]], bv[b], gsb[b])
            if t >= 1:
                p = (t - 1) % 2
                row = (base + j0 + t - 1) * _K
                ga[t - 1].wait()
                wa[t - 1] = pltpu.async_copy(
                    av[p], a_h.at[pl.ds(row, _K)], wsa[p])
                gb[t - 1].wait()
                wb[t - 1] = pltpu.async_copy(
                    bv[p], b_h.at[pl.ds(row, _K)], wsb[p])
        last = _PGK - 1
        p = last % 2
        row = (base + j0 + last) * _K
        ga[last].wait()
        wa[last] = pltpu.async_copy(av[p], a_h.at[pl.ds(row, _K)], wsa[p])
        gb[last].wait()
        wb[last] = pltpu.async_copy(bv[p], b_h.at[pl.ds(row, _K)], wsb[p])
        wa[last - 1].wait()
        wb[last - 1].wait()
        wa[last].wait()
        wb[last].wait()
        return carry

    lax.fori_loop(0, _CHP // _PGK, group, 0)


def _pair_gather(h, srcp, dstp):
    out_type = [
        jax.ShapeDtypeStruct((_PTOT, _D), jnp.float32),
        jax.ShapeDtypeStruct((_PTOT, _D), jnp.float32),
    ]
    scratch = [
        pltpu.VMEM((_CHP, _K), jnp.int32),
        pltpu.VMEM((_CHP, _K), jnp.int32),
        pltpu.VMEM((_K, _D), jnp.float32),
        pltpu.VMEM((_K, _D), jnp.float32),
        pltpu.VMEM((_K, _D), jnp.float32),
        pltpu.VMEM((_K, _D), jnp.float32),
    ] + [pltpu.SemaphoreType.DMA] * 8
    f = pl.kernel(_pair_gather_body, out_type=out_type, mesh=_mesh(),
                  scratch_types=scratch)
    return f(h, srcp, dstp)


def _layer_tc(h, parts, degp, Ws, Wn, b, relu):
    R = 1000

    def body(h_ref, p_ref, d_ref, ws_ref, wn_ref, b_ref, o_ref):
        deg = jnp.maximum(d_ref[0, :, 0] + d_ref[1, :, 0], 1.0)
        agg = (p_ref[0] + p_ref[1]) / deg[:, None]
        o = (jnp.dot(h_ref[...], ws_ref[...], preferred_element_type=jnp.float32)
             + jnp.dot(agg, wn_ref[...], preferred_element_type=jnp.float32)
             + b_ref[...])
        if relu:
            o = jnp.maximum(o, 0.0)
        o_ref[...] = o

    return pl.pallas_call(
        body,
        grid=(_N // R,),
        in_specs=[
            pl.BlockSpec((R, _D), lambda i: (i, 0)),
            pl.BlockSpec((_NC, R, _D), lambda i: (0, i, 0)),
            pl.BlockSpec((_NC, R, _D), lambda i: (0, i, 0)),
            pl.BlockSpec((_D, _D), lambda i: (0, 0)),
            pl.BlockSpec((_D, _D), lambda i: (0, 0)),
            pl.BlockSpec((1, _D), lambda i: (0, 0)),
        ],
        out_specs=pl.BlockSpec((R, _D), lambda i: (i, 0)),
        out_shape=jax.ShapeDtypeStruct((_N, _D), jnp.float32),
    )(h, parts, degp, Ws, Wn, b.reshape(1, _D))


def _mlp_tc(a, b, P0, pb0, P1, pb1, P2, pb2):
    R = 1024

    def body(a_ref, b_ref, p0, q0, p1, q1, p2, q2, o_ref):
        z = a_ref[...] * b_ref[...]
        z = jnp.maximum(
            jnp.dot(z, p0[...], preferred_element_type=jnp.float32) + q0[...], 0.0)
        z = jnp.maximum(
            jnp.dot(z, p1[...], preferred_element_type=jnp.float32) + q1[...], 0.0)
        o_ref[...] = jnp.dot(z, p2[...], preferred_element_type=jnp.float32) + q2[...]

    return pl.pallas_call(
        body,
        grid=(_PTOT // R,),
        in_specs=[
            pl.BlockSpec((R, _D), lambda i: (i, 0)),
            pl.BlockSpec((R, _D), lambda i: (i, 0)),
            pl.BlockSpec((_D, _D), lambda i: (0, 0)),
            pl.BlockSpec((1, _D), lambda i: (0, 0)),
            pl.BlockSpec((_D, _D), lambda i: (0, 0)),
            pl.BlockSpec((1, _D), lambda i: (0, 0)),
            pl.BlockSpec((_D, 1), lambda i: (0, 0)),
            pl.BlockSpec((1, 1), lambda i: (0, 0)),
        ],
        out_specs=pl.BlockSpec((R, 1), lambda i: (i, 0)),
        out_shape=jax.ShapeDtypeStruct((_PTOT, 1), jnp.float32),
    )(a, b, P0, pb0.reshape(1, _D), P1, pb1.reshape(1, _D), P2,
      pb2.reshape(1, 1))


def kernel(x, edge_index, pos_src, pos_dst, neg_src, neg_dst,
           W_self0, W_neigh0, b0, W_self1, W_neigh1, b1, W_self2, W_neigh2, b2,
           P0, pb0, P1, pb1, P2, pb2):
    src = edge_index[0]
    dst = edge_index[1]
    pad_e = _NW * _CH * _K - _E
    spad = jnp.concatenate([src, jnp.zeros((pad_e,), jnp.int32)])
    dpad = jnp.concatenate([dst, jnp.full((pad_e,), _N, jnp.int32)])
    srcp = spad.reshape(_NW, _CHA, _KA)
    dstp = dpad.reshape(_NW, _CHA, _KA)
    dstp_deg = dpad.reshape(_NW, _CH, _K)
    z = jnp.zeros((_K, _D), jnp.float32)
    ones_k = jnp.ones((_K, _D), jnp.float32)

    degp = _deg_agg(dstp_deg, z, ones_k)

    parts = _sage_agg(x, srcp, dstp, z)
    h1 = _layer_tc(x, parts, degp, W_self0, W_neigh0, b0, relu=True)
    parts = _sage_agg(h1, srcp, dstp, z)
    h2 = _layer_tc(h1, parts, degp, W_self1, W_neigh1, b1, relu=True)
    parts = _sage_agg(h2, srcp, dstp, z)
    h3 = _layer_tc(h2, parts, degp, W_self2, W_neigh2, b2, relu=False)

    pad_p = _EPPAD - _EP
    zp = jnp.zeros((pad_p,), jnp.int32)
    ps = jnp.concatenate(
        [pos_src, zp, neg_src, zp]).reshape(_NW, _CHP, _K)
    pd = jnp.concatenate(
        [pos_dst, zp, neg_dst, zp]).reshape(_NW, _CHP, _K)
    a, bm = _pair_gather(h3, ps, pd)
    scores = _mlp_tc(a, bm, P0, pb0, P1, pb1, P2, pb2)
    return scores[:_EP], scores[_EPPAD:_EPPAD + _EP]
